# baseline (speedup 1.0000x reference)
import functools

import jax
import jax.numpy as jnp
from jax import lax
from jax.experimental import pallas as pl
from jax.experimental.pallas import tpu as pltpu
from jax.experimental.pallas import tpu_sc as plsc

V = 10000
D = 128
K = 32
F = 64

NC = 2
NS = 16
L = 16
NW = NC * NS
VP = 10240
RPW = VP // NW
CH = 2
NSLOT = 4
GB = CH * K
NCHUNK = RPW // CH


def _tc_body(x_ref, f_ref, d_ref, wdx_ref, wdf_ref, bd_ref, wf_ref, bf_ref,
             w_ref, feat_ref, dn_ref):
    xb = x_ref[...]
    fb = f_ref[...]
    s = (jnp.sum(xb * wdx_ref[...], axis=1, keepdims=True)
         + jnp.sum(fb * wdf_ref[...], axis=1, keepdims=True)
         + bd_ref[0, 0])
    scale = 10.0 / (1.0 + jnp.exp(-s))
    dn = d_ref[...] * scale
    dn_ref[...] = dn
    w_ref[...] = jnp.exp(-10.0 * dn)
    feat_ref[...] = jnp.maximum(
        jnp.dot(fb, wf_ref[...], preferred_element_type=jnp.float32)
        + bf_ref[...], 0.0)


def _tc_stage(x, featin, d, wdx, wdf, bd, wf, bf):
    bv = 1000
    grid = (V // bv,)
    return pl.pallas_call(
        _tc_body,
        grid=grid,
        in_specs=[
            pl.BlockSpec((bv, D), lambda i: (i, 0)),
            pl.BlockSpec((bv, D), lambda i: (i, 0)),
            pl.BlockSpec((bv, K), lambda i: (i, 0)),
            pl.BlockSpec((1, D), lambda i: (0, 0)),
            pl.BlockSpec((1, D), lambda i: (0, 0)),
            pl.BlockSpec((1, 1), lambda i: (0, 0)),
            pl.BlockSpec((D, F), lambda i: (0, 0)),
            pl.BlockSpec((1, F), lambda i: (0, 0)),
        ],
        out_specs=[
            pl.BlockSpec((bv, K), lambda i: (i, 0)),
            pl.BlockSpec((bv, F), lambda i: (i, 0)),
            pl.BlockSpec((bv, K), lambda i: (i, 0)),
        ],
        out_shape=[
            jax.ShapeDtypeStruct((V, K), jnp.float32),
            jax.ShapeDtypeStruct((V, F), jnp.float32),
            jax.ShapeDtypeStruct((V, K), jnp.float32),
        ],
    )(x, featin, d, wdx, wdf, bd, wf, bf)


def _sc_body(featp_hbm, nidxf_hbm, wflat_hbm, out_hbm,
             idx_all, poff_all, w_all, tab_sh,
             rows0, rows1, rows2, rows3, own0, own1, own2, own3,
             out0, out1, out2, out3,
             semg0, semg1, semg2, semg3, semn0, semn1, semn2, semn3,
             semo0, semo1, semo2, semo3):
    sid = lax.axis_index("s")
    wid = sid * NC + lax.axis_index("c")
    base = wid * RPW

    stripe = (VP // 2) // NS
    pltpu.sync_copy(featp_hbm.at[pl.ds(sid * stripe, stripe)],
                    tab_sh.at[pl.ds(sid * stripe, stripe)])
    pltpu.sync_copy(nidxf_hbm.at[pl.ds(base * K, RPW * K)], idx_all)
    pltpu.sync_copy(wflat_hbm.at[pl.ds(base * K, RPW * K)], w_all)

    def prep(j, carry):
        v = idx_all[pl.ds(j * L, L)]
        poff_all[pl.ds(j * L, L)] = (v & 1) * F
        idx_all[pl.ds(j * L, L)] = v >> 1
        return carry

    lax.fori_loop(0, RPW * K // L, prep, 0)
    plsc.subcore_barrier()

    slots = ((rows0, own0, out0, semg0, semn0, semo0),
             (rows1, own1, out1, semg1, semn1, semo1),
             (rows2, own2, out2, semg2, semn2, semo2),
             (rows3, own3, out3, semg3, semn3, semo3))

    def fire(c, slot):
        rows_v, own_v, _, semg, semn, _ = slots[slot]
        src = tab_sh if slot % 2 == 0 else featp_hbm
        pltpu.async_copy(src.at[idx_all.at[pl.ds(c * GB, GB)]],
                         rows_v, semg)
        pltpu.async_copy(
            tab_sh.at[pl.ds((base + c * CH) // 2, CH // 2)], own_v, semn)

    for s0 in range(NSLOT):
        fire(s0, s0)

    def pair(i, carry):
        for slot in range(NSLOT):
            rows_v, own_v, out_v, semg, semn, semo = slots[slot]
            c = NSLOT * i + slot
            srcw = tab_sh if slot % 2 == 0 else featp_hbm
            pltpu.make_async_copy(
                srcw.at[idx_all.at[pl.ds(0, GB)]], rows_v, semg).wait()
            pltpu.make_async_copy(
                tab_sh.at[pl.ds(0, CH // 2)], own_v, semn).wait()

            @pl.when(i > 0)
            def _():
                pltpu.make_async_copy(
                    out_v, out_hbm.at[pl.ds(0, CH)], semo).wait()

            for dl in range(CH):
                row0 = dl * K
                accs = [jnp.zeros((L,), jnp.float32) for _ in range(F // L)]
                accm = [jnp.full((L,), -jnp.inf, jnp.float32)
                        for _ in range(F // L)]
                for kg in range(K // L):
                    w16 = w_all[pl.ds(c * GB + row0 + kg * L, L)]
                    p16 = poff_all[pl.ds(c * GB + row0 + kg * L, L)]
                    for kl in range(L):
                        k = kg * L + kl
                        wv = jnp.full((L,), w16[kl])
                        p = p16[kl]
                        for t in range(F // L):
                            nf = rows_v[row0 + k, pl.ds(p + t * L, L)]
                            wfv = wv * nf
                            accs[t] = accs[t] + wfv
                            accm[t] = jnp.maximum(accm[t], wfv)
                for t in range(F // L):
                    ov = own_v[dl // 2, pl.ds((dl % 2) * F + t * L, L)]
                    out_v[dl, pl.ds(t * L, L)] = accs[t] * (1.0 / K) - ov
                    out_v[dl, pl.ds(F + t * L, L)] = accm[t] - ov

            pltpu.async_copy(out_v, out_hbm.at[pl.ds(base + c * CH, CH)],
                             semo)

            @pl.when(c + NSLOT < NCHUNK)
            def _():
                fire(c + NSLOT, slot)
        return carry

    lax.fori_loop(0, NCHUNK // NSLOT, pair, 0)

    for ob, so in ((out0, semo0), (out1, semo1), (out2, semo2),
                   (out3, semo3)):
        pltpu.make_async_copy(ob, out_hbm.at[pl.ds(0, CH)], so).wait()


_sc_knn = functools.partial(
    pl.kernel,
    out_type=jax.ShapeDtypeStruct((VP, 2 * F), jnp.float32),
    mesh=plsc.VectorSubcoreMesh(
        core_axis_name="c", subcore_axis_name="s",
        num_cores=NC, num_subcores=NS),
    scratch_types=[
        pltpu.VMEM((RPW * K,), jnp.int32),
        pltpu.VMEM((RPW * K,), jnp.int32),
        pltpu.VMEM((RPW * K,), jnp.float32),
        pltpu.VMEM_SHARED((VP // 2, 2 * F), jnp.float32),
    ] + [pltpu.VMEM((GB, 2 * F), jnp.float32)] * 4
      + [pltpu.VMEM((CH // 2, 2 * F), jnp.float32)] * 4
      + [pltpu.VMEM((CH, 2 * F), jnp.float32)] * 4
      + [pltpu.SemaphoreType.DMA] * 12,
)(_sc_body)


def kernel(x, neighbor_indices, distancesq,
           Wd0, bd0, Wf0, bf0,
           Wd1, bd1, Wf1, bf1,
           Wd2, bd2, Wf2, bf2):
    zcol = jnp.zeros((1, D), jnp.float32)
    wdx = [Wd0[:, 0].reshape(1, D), Wd1[:D, 0].reshape(1, D),
           Wd2[:D, 0].reshape(1, D)]
    wdf = [zcol, Wd1[D:, 0].reshape(1, D), Wd2[D:, 0].reshape(1, D)]
    bd = [bd0.reshape(1, 1), bd1.reshape(1, 1), bd2.reshape(1, 1)]
    wfm = [Wf0, Wf1, Wf2]
    bfv = [bf0.reshape(1, F), bf1.reshape(1, F), bf2.reshape(1, F)]

    nidx_flat = jnp.pad(neighbor_indices, ((0, VP - V), (0, 0))).reshape(-1)

    outs = []
    featin = x
    d = distancesq
    for i in range(3):
        w, feat, d = _tc_stage(x, featin, d, wdx[i], wdf[i], bd[i],
                               wfm[i], bfv[i])
        feat_p = jnp.pad(feat, ((0, VP - V), (0, 0))).reshape(VP // 2, 2 * F)
        w_flat = jnp.pad(w, ((0, VP - V), (0, 0))).reshape(-1)
        out_p = _sc_knn(feat_p, nidx_flat, w_flat)
        out_i = out_p[:V]
        outs.append(out_i)
        featin = out_i
    return jnp.concatenate(outs + [x], axis=-1)


# speedup vs baseline: 2.5595x; 2.5595x over previous
import functools

import jax
import jax.numpy as jnp
from jax import lax
from jax.experimental import pallas as pl
from jax.experimental.pallas import tpu as pltpu
from jax.experimental.pallas import tpu_sc as plsc

V = 10000
D = 128
K = 32
F = 64

NC = 2
NS = 16
L = 16
NW = NC * NS
VP = 10240
RPW = VP // NW
CH = 2
NSLOT = 4
GB = CH * K
NCHUNK = RPW // CH


def _tc_body(xp_ref, fp_ref, dp_ref, wd_ref, bd_ref, wf_ref, bf_ref,
             w_ref, feat_ref, dn_ref):
    xp = xp_ref[...]
    fp = fp_ref[...]
    wdc = wd_ref[...]
    top = wdc[:D, :]
    bot = wdc[D:, :]
    z1 = jnp.zeros((D, 1), jnp.float32)
    wd2x = jnp.concatenate(
        [jnp.concatenate([top, z1], 0), jnp.concatenate([z1, top], 0)], 1)
    wd2f = jnp.concatenate(
        [jnp.concatenate([bot, z1], 0), jnp.concatenate([z1, bot], 0)], 1)
    s2 = (jnp.dot(xp, wd2x, preferred_element_type=jnp.float32)
          + jnp.dot(fp, wd2f, preferred_element_type=jnp.float32)
          + bd_ref[0, 0])
    scale2 = 10.0 / (1.0 + jnp.exp(-s2))
    dp = dp_ref[...]
    dnp = jnp.concatenate([dp[:, :K] * scale2[:, 0:1],
                           dp[:, K:] * scale2[:, 1:2]], 1)
    dn_ref[...] = dnp
    w_ref[...] = jnp.exp(-10.0 * dnp)
    wfm = wf_ref[...]
    zf = jnp.zeros((D, F), jnp.float32)
    w2 = jnp.concatenate([jnp.concatenate([wfm, zf], 1),
                          jnp.concatenate([zf, wfm], 1)], 0)
    bf2 = jnp.concatenate([bf_ref[...], bf_ref[...]], 1)
    feat_ref[...] = jnp.maximum(
        jnp.dot(fp, w2, preferred_element_type=jnp.float32) + bf2, 0.0)


def _tc_stage(xp, fp, dp, wd, bd, wf, bf):
    bv2 = 1000
    grid = (V // (2 * bv2),)
    return pl.pallas_call(
        _tc_body,
        grid=grid,
        in_specs=[
            pl.BlockSpec((bv2, 2 * D), lambda i: (i, 0)),
            pl.BlockSpec((bv2, 2 * D), lambda i: (i, 0)),
            pl.BlockSpec((bv2, 2 * K), lambda i: (i, 0)),
            pl.BlockSpec((2 * D, 1), lambda i: (0, 0)),
            pl.BlockSpec((1, 1), lambda i: (0, 0)),
            pl.BlockSpec((D, F), lambda i: (0, 0)),
            pl.BlockSpec((1, F), lambda i: (0, 0)),
        ],
        out_specs=[
            pl.BlockSpec((bv2, 2 * K), lambda i: (i, 0)),
            pl.BlockSpec((bv2, 2 * F), lambda i: (i, 0)),
            pl.BlockSpec((bv2, 2 * K), lambda i: (i, 0)),
        ],
        out_shape=[
            jax.ShapeDtypeStruct((VP // 2, 2 * K), jnp.float32),
            jax.ShapeDtypeStruct((VP // 2, 2 * F), jnp.float32),
            jax.ShapeDtypeStruct((V // 2, 2 * K), jnp.float32),
        ],
    )(xp, fp, dp, wd, bd, wf, bf)


def _sc_body(featp_hbm, nidxf_hbm, wflat_hbm, out_hbm,
             idx_all, poff_all, w_all, tab_sh,
             rows0, rows1, rows2, rows3, own0, own1, own2, own3,
             out0, out1, out2, out3,
             semg0, semg1, semg2, semg3, semn0, semn1, semn2, semn3,
             semo0, semo1, semo2, semo3):
    sid = lax.axis_index("s")
    wid = sid * NC + lax.axis_index("c")
    base = wid * RPW

    stripe = (VP // 2) // NS
    pltpu.sync_copy(featp_hbm.at[pl.ds(sid * stripe, stripe)],
                    tab_sh.at[pl.ds(sid * stripe, stripe)])
    pltpu.sync_copy(nidxf_hbm.at[pl.ds(base * K, RPW * K)], idx_all)
    pltpu.sync_copy(wflat_hbm.at[pl.ds(base * K, RPW * K)], w_all)

    def prep(j, carry):
        v = idx_all[pl.ds(j * L, L)]
        poff_all[pl.ds(j * L, L)] = (v & 1) * F
        idx_all[pl.ds(j * L, L)] = v >> 1
        return carry

    lax.fori_loop(0, RPW * K // L, prep, 0)
    plsc.subcore_barrier()

    slots = ((rows0, own0, out0, semg0, semn0, semo0),
             (rows1, own1, out1, semg1, semn1, semo1),
             (rows2, own2, out2, semg2, semn2, semo2),
             (rows3, own3, out3, semg3, semn3, semo3))

    def fire(c, slot):
        rows_v, own_v, _, semg, semn, _ = slots[slot]
        pltpu.async_copy(tab_sh.at[idx_all.at[pl.ds(c * GB, GB)]],
                         rows_v, semg)
        pltpu.async_copy(
            tab_sh.at[pl.ds((base + c * CH) // 2, CH // 2)], own_v, semn)

    for s0 in range(NSLOT):
        fire(s0, s0)

    def pair(i, carry):
        for slot in range(NSLOT):
            rows_v, own_v, out_v, semg, semn, semo = slots[slot]
            c = NSLOT * i + slot
            pltpu.make_async_copy(
                tab_sh.at[idx_all.at[pl.ds(0, GB)]], rows_v, semg).wait()
            pltpu.make_async_copy(
                tab_sh.at[pl.ds(0, CH // 2)], own_v, semn).wait()

            @pl.when(i > 0)
            def _():
                pltpu.make_async_copy(
                    out_v, out_hbm.at[pl.ds(0, CH)], semo).wait()

            for dl in range(CH):
                row0 = dl * K
                accs = [jnp.zeros((L,), jnp.float32) for _ in range(F // L)]
                accm = [jnp.full((L,), -jnp.inf, jnp.float32)
                        for _ in range(F // L)]
                for kg in range(K // L):
                    w16 = w_all[pl.ds(c * GB + row0 + kg * L, L)]
                    p16 = poff_all[pl.ds(c * GB + row0 + kg * L, L)]
                    for kl in range(L):
                        k = kg * L + kl
                        wv = jnp.full((L,), w16[kl])
                        p = p16[kl]
                        for t in range(F // L):
                            nf = rows_v[row0 + k, pl.ds(p + t * L, L)]
                            wfv = wv * nf
                            accs[t] = accs[t] + wfv
                            accm[t] = jnp.maximum(accm[t], wfv)
                for t in range(F // L):
                    ov = own_v[dl // 2, pl.ds((dl % 2) * F + t * L, L)]
                    out_v[dl, pl.ds(t * L, L)] = accs[t] * (1.0 / K) - ov
                    out_v[dl, pl.ds(F + t * L, L)] = accm[t] - ov

            pltpu.async_copy(out_v, out_hbm.at[pl.ds(base + c * CH, CH)],
                             semo)

            @pl.when(c + NSLOT < NCHUNK)
            def _():
                fire(c + NSLOT, slot)
        return carry

    lax.fori_loop(0, NCHUNK // NSLOT, pair, 0)

    for ob, so in ((out0, semo0), (out1, semo1), (out2, semo2),
                   (out3, semo3)):
        pltpu.make_async_copy(ob, out_hbm.at[pl.ds(0, CH)], so).wait()


_sc_knn = functools.partial(
    pl.kernel,
    out_type=jax.ShapeDtypeStruct((VP, 2 * F), jnp.float32),
    mesh=plsc.VectorSubcoreMesh(
        core_axis_name="c", subcore_axis_name="s",
        num_cores=NC, num_subcores=NS),
    scratch_types=[
        pltpu.VMEM((RPW * K,), jnp.int32),
        pltpu.VMEM((RPW * K,), jnp.int32),
        pltpu.VMEM((RPW * K,), jnp.float32),
        pltpu.VMEM_SHARED((VP // 2, 2 * F), jnp.float32),
    ] + [pltpu.VMEM((GB, 2 * F), jnp.float32)] * 4
      + [pltpu.VMEM((CH // 2, 2 * F), jnp.float32)] * 4
      + [pltpu.VMEM((CH, 2 * F), jnp.float32)] * 4
      + [pltpu.SemaphoreType.DMA] * 12,
)(_sc_body)


def kernel(x, neighbor_indices, distancesq,
           Wd0, bd0, Wf0, bf0,
           Wd1, bd1, Wf1, bf1,
           Wd2, bd2, Wf2, bf2):
    wd = [jnp.concatenate([Wd0, jnp.zeros((D, 1), jnp.float32)], 0),
          Wd1, Wd2]
    bd = [bd0.reshape(1, 1), bd1.reshape(1, 1), bd2.reshape(1, 1)]
    wfm = [Wf0, Wf1, Wf2]
    bfv = [bf0.reshape(1, F), bf1.reshape(1, F), bf2.reshape(1, F)]

    nidx_flat = jnp.pad(neighbor_indices, ((0, VP - V), (0, 0))).reshape(-1)
    xp = x.reshape(V // 2, 2 * D)
    dp = distancesq.reshape(V // 2, 2 * K)

    outs = []
    fp = xp
    for i in range(3):
        w2, featp, dp = _tc_stage(xp, fp, dp, wd[i], bd[i], wfm[i], bfv[i])
        w_flat = w2.reshape(-1)
        out_p = _sc_knn(featp, nidx_flat, w_flat)
        outs.append(out_p[:V])
        if i < 2:
            fp = out_p.reshape(VP // 2, 2 * D)[:V // 2]
    return jnp.concatenate(outs + [x], axis=-1)
